# baseline (speedup 1.0000x reference)
import functools

import jax
import jax.numpy as jnp
from jax import lax
from jax.experimental import pallas as pl
from jax.experimental.pallas import tpu as pltpu
from jax.experimental.pallas import tpu_sc as plsc

B, C, T = 4, 256, 1024
K = 3
F32 = jnp.float32


def _mm(x, w):
    return lax.dot_general(x, w, (((1,), (1,)), ((), ())),
                           preferred_element_type=F32)


_PAR = pltpu.CompilerParams(dimension_semantics=("parallel",))


def _block_diag(w, groups):
    o, ig = w.shape
    og = o // groups
    rows = jnp.arange(o)[:, None] // og
    cols = jnp.arange(groups * ig)[None, :] // ig
    return jnp.tile(w, (1, groups)) * (rows == cols).astype(w.dtype)


def _top3(x, valid_ref, b, i_ref):
    g = lax.dot_general(x, x, (((1,), (1,)), ((), ())),
                        preferred_element_type=F32)
    xx = jnp.sum(x * x, axis=1)
    pd = 2.0 * g - xx[:, None] - xx[None, :]
    pd = jnp.where(valid_ref[0] > 0.5, pd, -1e9)
    iota_c = lax.broadcasted_iota(jnp.int32, (T, T), 1)
    offs = b * T
    for j in range(K):
        m = jnp.max(pd, axis=1)
        idxj = jnp.min(jnp.where(pd == m[:, None], iota_c, T), axis=1)
        i_ref[j, 0, 0] = idxj + offs
        if j < K - 1:
            pd = jnp.where(iota_c == idxj[:, None], -jnp.inf, pd)


def _spath(yg_refs, ybp, tpi_ref, w2_ref, w3_ref, b2_ref):
    smax = None
    for j in range(K):
        s1 = jax.nn.relu(yg_refs[j][...] + ybp)
        s2 = jax.nn.relu(_mm(s1, w2_ref[...]) + b2_ref[...])
        s3 = _mm(s2, w3_ref[...])
        smax = s3 if smax is None else jnp.maximum(smax, s3)
    return jax.nn.relu(tpi_ref[0] + smax)


def _full(shape):
    return pl.BlockSpec(shape, lambda b: (0,) * len(shape))


def _bspec(*shape):
    return pl.BlockSpec((1,) + tuple(shape), lambda b: (b,) + (0,) * len(shape))


def _yg_specs():
    def mk(j):
        return pl.BlockSpec((T, 128), lambda b: (j * B + b, 0))
    return [mk(j) for j in range(K)]


_GW = [
    _full((128, 256)), _full((128, 128)), _full((128, 128)),
    _full((128, 128)), _full((256, 128)),
    _full((128, 256)), _full((128, 256)), _full((128, 128)),
    _full((256, 128)),
    _full((1, 128)), _full((1, 128)), _full((1, 256)), _full((1, 128)),
    _full((1, 128)),
]


def _prep_block(p):
    return (
        p['wt1'][:, :, 0],
        _block_diag(p['wt2'][:, :, 0], 32),
        _block_diag(p['wt2'][:, :, 1], 32),
        _block_diag(p['wt2'][:, :, 2], 32),
        p['wt3'][:, :, 0],
        p['ws1'][:, :C, 0, 0], p['ws1'][:, C:, 0, 0],
        _block_diag(p['ws2'][:, :, 0, 0], 32), p['ws3'][:, :, 0, 0],
        p['bt1'].reshape(1, 128), p['bt2'].reshape(1, 128),
        (p['bt3'] + p['bs3']).reshape(1, 256),
        p['bs1'].reshape(1, 128), p['bs2'].reshape(1, 128),
    )


def _block_tail(x, b, valid_ref, gw_refs, tpi_ref, ya_ref, ybp_ref, i_ref):
    (wt1, wt2a, wt2b, wt2c, wt3, wa, wbs, _w2, _w3,
     bt1, bt2, bt3, bs1, _b2) = gw_refs
    tpi_ref[0] = x + _tpath(x, wt1, (wt2a, wt2b, wt2c), wt3, bt1, bt2, bt3)
    ya_ref[0] = _mm(x, wa[...])
    ybp_ref[0] = _mm(x, wbs[...]) + bs1[...]
    _top3(x, valid_ref, b, i_ref)


def _tpath(x, wt1_ref, wt2_refs, wt3_ref, bt1_ref, bt2_ref, bt3_ref):
    t1 = jax.nn.relu(_mm(x, wt1_ref[...]) + bt1_ref[...])
    z = jnp.zeros((1, 128), F32)
    t1p = jnp.concatenate([z, t1, z], axis=0)
    t2 = _mm(t1p[0:T], wt2_refs[0][...])
    t2 += _mm(t1p[1:T + 1], wt2_refs[1][...])
    t2 += _mm(t1p[2:T + 2], wt2_refs[2][...])
    t2 = jax.nn.relu(t2 + bt2_ref[...])
    return _mm(t2, wt3_ref[...]) + bt3_ref[...]


def _kp1_body(eye_ref, x_ref, valid_ref, wb0_ref, wb1_ref, wb2_ref, bb_ref,
              *rest):
    gw_refs = rest[:14]
    tpi_ref, ya_ref, ybp_ref, i_ref = rest[14:]
    b = pl.program_id(0)
    xt = lax.dot_general(eye_ref[...], x_ref[0], (((1,), (1,)), ((), ())),
                         precision=lax.Precision.HIGHEST,
                         preferred_element_type=F32)
    z = jnp.zeros((1, C), F32)
    xp = jnp.concatenate([z, xt, z], axis=0)
    acc = _mm(xp[0:T], wb0_ref[...])
    acc += _mm(xp[1:T + 1], wb1_ref[...])
    acc += _mm(xp[2:T + 2], wb2_ref[...])
    x = jax.nn.relu(acc + bb_ref[...])
    _block_tail(x, b, valid_ref, gw_refs, tpi_ref, ya_ref, ybp_ref, i_ref)


def _out_shapes():
    return [
        jax.ShapeDtypeStruct((B, T, C), F32),
        jax.ShapeDtypeStruct((B, T, 128), F32),
        jax.ShapeDtypeStruct((B, T, 128), F32),
        jax.ShapeDtypeStruct((K, B, 1, T), jnp.int32),
    ]


def _out_specs():
    return [_bspec(T, C), _bspec(T, 128), _bspec(T, 128),
            pl.BlockSpec((K, 1, 1, T), lambda b: (0, b, 0, 0))]


def _kp1(eye, snip, valid, wb0, wb1, wb2, bb, gw):
    return pl.pallas_call(
        _kp1_body,
        grid=(B,),
        in_specs=[_full((T, T)), _bspec(C, T), _bspec(1, T), _full((C, C)),
                  _full((C, C)), _full((C, C)), _full((1, C))] + _GW,
        out_specs=_out_specs(),
        out_shape=_out_shapes(),
        compiler_params=_PAR,
    )(eye, snip, valid, wb0, wb1, wb2, bb, *gw)


def _kp2_body(g0_ref, g1_ref, g2_ref, ybp1_ref, tpi1_ref, valid_ref,
              w2p_ref, w3p_ref, b2p_ref, *rest):
    gw_refs = rest[:14]
    tpi_ref, ya_ref, ybp_ref, i_ref = rest[14:]
    b = pl.program_id(0)
    x = _spath((g0_ref, g1_ref, g2_ref), ybp1_ref[0], tpi1_ref,
               w2p_ref, w3p_ref, b2p_ref)
    _block_tail(x, b, valid_ref, gw_refs, tpi_ref, ya_ref, ybp_ref, i_ref)


def _kp2(g, ybp1, tpi1, valid, w2p, w3p, b2p, gw):
    return pl.pallas_call(
        _kp2_body,
        grid=(B,),
        in_specs=_yg_specs() + [
            _bspec(T, 128), _bspec(T, C), _bspec(1, T),
            _full((128, 128)), _full((256, 128)), _full((1, 128)),
        ] + _GW,
        out_specs=_out_specs(),
        out_shape=_out_shapes(),
        compiler_params=_PAR,
    )(g, g, g, ybp1, tpi1, valid, w2p, w3p, b2p, *gw)


def _kp3_body(eye_ref, g0_ref, g1_ref, g2_ref, ybp_ref, tpi_ref,
              w2_ref, w3_ref, b2_ref, out_ref):
    x = _spath((g0_ref, g1_ref, g2_ref), ybp_ref[0], tpi_ref,
               w2_ref, w3_ref, b2_ref)
    out_ref[0] = lax.dot_general(x, eye_ref[...], (((0,), (0,)), ((), ())),
                                 preferred_element_type=F32)


def _kp3(eye, g, ybp, tpi, w2, w3, b2):
    return pl.pallas_call(
        _kp3_body,
        grid=(B,),
        in_specs=[_full((T, T))] + _yg_specs() + [
            _bspec(T, 128), _bspec(T, C),
            _full((128, 128)), _full((256, 128)), _full((1, 128)),
        ],
        out_specs=_bspec(C, T),
        out_shape=jax.ShapeDtypeStruct((B, C, T), F32),
        compiler_params=_PAR,
    )(eye, g, g, g, ybp, tpi, w2, w3, b2)


_NG = B * T * K
_D = 128


def _sc_gather(table, idx):
    info = plsc.get_sparse_core_info()
    nc, ns = info.num_cores, info.num_subcores
    nw = nc * ns
    b_per_w = _NG // nw
    mesh = plsc.VectorSubcoreMesh(core_axis_name="c", subcore_axis_name="s")

    nch = 4
    ch = b_per_w // nch

    @functools.partial(
        pl.kernel, mesh=mesh,
        out_type=jax.ShapeDtypeStruct((_NG, _D), F32),
        scratch_types=[
            pltpu.VMEM((b_per_w,), jnp.int32),
            pltpu.VMEM((nch, ch, _D), F32),
            pltpu.SemaphoreType.DMA,
        ] + [pltpu.SemaphoreType.DMA] * nch,
    )
    def gk(table_hbm, idx_hbm, out_hbm, idx_v, rows_v, wsem, *gsems):
        wid = lax.axis_index("s") * nc + lax.axis_index("c")
        base = wid * b_per_w
        pltpu.sync_copy(idx_hbm.at[pl.ds(base, b_per_w)], idx_v)
        gds = [
            pltpu.async_copy(table_hbm.at[idx_v.at[pl.ds(i * ch, ch)]],
                             rows_v.at[i], gsems[i])
            for i in range(nch)
        ]
        wds = []
        for i in range(nch):
            gds[i].wait()
            wds.append(pltpu.async_copy(
                rows_v.at[i], out_hbm.at[pl.ds(base + i * ch, ch)], wsem))
        for w in wds:
            w.wait()

    return gk(table, idx)


def kernel(snip_feature, seg_lens, params):
    wb0 = _block_diag(params['w_b'][:, :, 0], 4)
    wb1 = _block_diag(params['w_b'][:, :, 1], 4)
    wb2 = _block_diag(params['w_b'][:, :, 2], 4)
    bb = params['b_b'].reshape(1, C)

    valid = (jnp.arange(T)[None, :] < seg_lens[:, None]).astype(F32)
    valid = valid.reshape(B, 1, T)

    g1 = _prep_block(params['g1'])
    g2 = _prep_block(params['g2'])
    eye = jnp.eye(T, dtype=F32)

    tpi1, ya1, ybp1, idx1 = _kp1(eye, snip_feature, valid, wb0, wb1, wb2,
                                 bb, g1)
    yg1 = _sc_gather(ya1.reshape(B * T, 128), idx1.reshape(-1))
    tpi2, ya2, ybp2, idx2 = _kp2(
        yg1, ybp1, tpi1, valid, g1[7], g1[8], g1[13], g2)
    yg2 = _sc_gather(ya2.reshape(B * T, 128), idx2.reshape(-1))
    return _kp3(eye, yg2, ybp2, tpi2, g2[7], g2[8], g2[13])


# speedup vs baseline: 1.1344x; 1.1344x over previous
import functools

import jax
import jax.numpy as jnp
from jax import lax
from jax.experimental import pallas as pl
from jax.experimental.pallas import tpu as pltpu
from jax.experimental.pallas import tpu_sc as plsc

B, C, T = 4, 256, 1024
K = 3
F32 = jnp.float32


def _mm(x, w):
    return lax.dot_general(x, w, (((1,), (1,)), ((), ())),
                           preferred_element_type=F32)


def _mmb(x, w):
    return lax.dot_general(x.astype(jnp.bfloat16), w.astype(jnp.bfloat16),
                           (((1,), (1,)), ((), ())),
                           preferred_element_type=F32)


_PAR = pltpu.CompilerParams(dimension_semantics=("parallel",))


def _block_diag(w, groups):
    o, ig = w.shape
    og = o // groups
    rows = jnp.arange(o)[:, None] // og
    cols = jnp.arange(groups * ig)[None, :] // ig
    return jnp.tile(w, (1, groups)) * (rows == cols).astype(w.dtype)


def _top3(x, valid_ref, b, i_ref):
    g = lax.dot_general(x, x, (((1,), (1,)), ((), ())),
                        preferred_element_type=F32)
    xx = jnp.sum(x * x, axis=1)
    pd = 2.0 * g - xx[:, None] - xx[None, :]
    pd = jnp.where(valid_ref[0] > 0.5, pd, -1e9)
    iota_c = lax.broadcasted_iota(jnp.int32, (T, T), 1)
    offs = b * T
    for j in range(K):
        m = jnp.max(pd, axis=1)
        idxj = jnp.min(jnp.where(pd == m[:, None], iota_c, T), axis=1)
        i_ref[j, 0, 0] = idxj + offs
        if j < K - 1:
            pd = jnp.where(iota_c == idxj[:, None], -jnp.inf, pd)


def _spath(yg_refs, ybp, tpi_ref, w2_ref, w3_ref, b2_ref):
    smax = None
    for j in range(K):
        s1 = jax.nn.relu(yg_refs[j][...] + ybp)
        s2 = jax.nn.relu(_mmb(s1, w2_ref[...]) + b2_ref[...])
        s3 = _mmb(s2, w3_ref[...])
        smax = s3 if smax is None else jnp.maximum(smax, s3)
    return jax.nn.relu(tpi_ref[0] + smax)


def _full(shape):
    return pl.BlockSpec(shape, lambda b: (0,) * len(shape))


def _bspec(*shape):
    return pl.BlockSpec((1,) + tuple(shape), lambda b: (b,) + (0,) * len(shape))


def _yg_specs():
    def mk(j):
        return pl.BlockSpec((T, 128), lambda b: (j * B + b, 0))
    return [mk(j) for j in range(K)]


_GW = [
    _full((128, 256)), _full((128, 128)), _full((128, 128)),
    _full((128, 128)), _full((256, 128)),
    _full((128, 256)), _full((128, 256)), _full((128, 128)),
    _full((256, 128)),
    _full((1, 128)), _full((1, 128)), _full((1, 256)), _full((1, 128)),
    _full((1, 128)),
]


def _prep_block(p):
    return (
        p['wt1'][:, :, 0],
        _block_diag(p['wt2'][:, :, 0], 32),
        _block_diag(p['wt2'][:, :, 1], 32),
        _block_diag(p['wt2'][:, :, 2], 32),
        p['wt3'][:, :, 0],
        p['ws1'][:, :C, 0, 0], p['ws1'][:, C:, 0, 0],
        _block_diag(p['ws2'][:, :, 0, 0], 32), p['ws3'][:, :, 0, 0],
        p['bt1'].reshape(1, 128), p['bt2'].reshape(1, 128),
        (p['bt3'] + p['bs3']).reshape(1, 256),
        p['bs1'].reshape(1, 128), p['bs2'].reshape(1, 128),
    )


def _block_tail(x, b, valid_ref, gw_refs, tpi_ref, ya_ref, ybp_ref, i_ref):
    (wt1, wt2a, wt2b, wt2c, wt3, wa, wbs, _w2, _w3,
     bt1, bt2, bt3, bs1, _b2) = gw_refs
    tpi_ref[0] = x + _tpath(x, wt1, (wt2a, wt2b, wt2c), wt3, bt1, bt2, bt3)
    ya_ref[0] = _mmb(x, wa[...])
    ybp_ref[0] = _mmb(x, wbs[...]) + bs1[...]
    _top3(x, valid_ref, b, i_ref)


def _tpath(x, wt1_ref, wt2_refs, wt3_ref, bt1_ref, bt2_ref, bt3_ref):
    t1 = jax.nn.relu(_mmb(x, wt1_ref[...]) + bt1_ref[...])
    z = jnp.zeros((1, 128), F32)
    t1p = jnp.concatenate([z, t1, z], axis=0)
    t2 = _mmb(t1p[0:T], wt2_refs[0][...])
    t2 += _mmb(t1p[1:T + 1], wt2_refs[1][...])
    t2 += _mmb(t1p[2:T + 2], wt2_refs[2][...])
    t2 = jax.nn.relu(t2 + bt2_ref[...])
    return _mmb(t2, wt3_ref[...]) + bt3_ref[...]


def _kp1_body(eye_ref, x_ref, valid_ref, wb0_ref, wb1_ref, wb2_ref, bb_ref,
              *rest):
    gw_refs = rest[:14]
    tpi_ref, ya_ref, ybp_ref, i_ref = rest[14:]
    b = pl.program_id(0)
    xt = lax.dot_general(eye_ref[...], x_ref[0], (((1,), (1,)), ((), ())),
                         preferred_element_type=F32)
    z = jnp.zeros((1, C), F32)
    xp = jnp.concatenate([z, xt, z], axis=0)
    acc = _mm(xp[0:T], wb0_ref[...])
    acc += _mm(xp[1:T + 1], wb1_ref[...])
    acc += _mm(xp[2:T + 2], wb2_ref[...])
    x = jax.nn.relu(acc + bb_ref[...])
    _block_tail(x, b, valid_ref, gw_refs, tpi_ref, ya_ref, ybp_ref, i_ref)


def _out_shapes():
    return [
        jax.ShapeDtypeStruct((B, T, C), F32),
        jax.ShapeDtypeStruct((B, T, 128), F32),
        jax.ShapeDtypeStruct((B, T, 128), F32),
        jax.ShapeDtypeStruct((K, B, 1, T), jnp.int32),
    ]


def _out_specs():
    return [_bspec(T, C), _bspec(T, 128), _bspec(T, 128),
            pl.BlockSpec((K, 1, 1, T), lambda b: (0, b, 0, 0))]


def _kp1(eye, snip, valid, wb0, wb1, wb2, bb, gw):
    return pl.pallas_call(
        _kp1_body,
        grid=(B,),
        in_specs=[_full((T, T)), _bspec(C, T), _bspec(1, T), _full((C, C)),
                  _full((C, C)), _full((C, C)), _full((1, C))] + _GW,
        out_specs=_out_specs(),
        out_shape=_out_shapes(),
        compiler_params=_PAR,
    )(eye, snip, valid, wb0, wb1, wb2, bb, *gw)


def _kp2_body(g0_ref, g1_ref, g2_ref, ybp1_ref, tpi1_ref, valid_ref,
              w2p_ref, w3p_ref, b2p_ref, *rest):
    gw_refs = rest[:14]
    tpi_ref, ya_ref, ybp_ref, i_ref = rest[14:]
    b = pl.program_id(0)
    x = _spath((g0_ref, g1_ref, g2_ref), ybp1_ref[0], tpi1_ref,
               w2p_ref, w3p_ref, b2p_ref)
    _block_tail(x, b, valid_ref, gw_refs, tpi_ref, ya_ref, ybp_ref, i_ref)


def _kp2(g, ybp1, tpi1, valid, w2p, w3p, b2p, gw):
    return pl.pallas_call(
        _kp2_body,
        grid=(B,),
        in_specs=_yg_specs() + [
            _bspec(T, 128), _bspec(T, C), _bspec(1, T),
            _full((128, 128)), _full((256, 128)), _full((1, 128)),
        ] + _GW,
        out_specs=_out_specs(),
        out_shape=_out_shapes(),
        compiler_params=_PAR,
    )(g, g, g, ybp1, tpi1, valid, w2p, w3p, b2p, *gw)


def _kp3_body(eye_ref, g0_ref, g1_ref, g2_ref, ybp_ref, tpi_ref,
              w2_ref, w3_ref, b2_ref, out_ref):
    x = _spath((g0_ref, g1_ref, g2_ref), ybp_ref[0], tpi_ref,
               w2_ref, w3_ref, b2_ref)
    out_ref[0] = lax.dot_general(x, eye_ref[...], (((0,), (0,)), ((), ())),
                                 preferred_element_type=F32)


def _kp3(eye, g, ybp, tpi, w2, w3, b2):
    return pl.pallas_call(
        _kp3_body,
        grid=(B,),
        in_specs=[_full((T, T))] + _yg_specs() + [
            _bspec(T, 128), _bspec(T, C),
            _full((128, 128)), _full((256, 128)), _full((1, 128)),
        ],
        out_specs=_bspec(C, T),
        out_shape=jax.ShapeDtypeStruct((B, C, T), F32),
        compiler_params=_PAR,
    )(eye, g, g, g, ybp, tpi, w2, w3, b2)


_NG = B * T * K
_D = 128


def _sc_gather(table, idx):
    info = plsc.get_sparse_core_info()
    nc, ns = info.num_cores, info.num_subcores
    nw = nc * ns
    b_per_w = _NG // nw
    mesh = plsc.VectorSubcoreMesh(core_axis_name="c", subcore_axis_name="s")

    @functools.partial(
        pl.kernel, mesh=mesh,
        out_type=jax.ShapeDtypeStruct((_NG, _D), F32),
        scratch_types=[
            pltpu.VMEM((b_per_w,), jnp.int32),
            pltpu.VMEM((b_per_w, _D), F32),
            pltpu.SemaphoreType.DMA,
        ],
    )
    def gk(table_hbm, idx_hbm, out_hbm, idx_v, rows_v, sem):
        wid = lax.axis_index("s") * nc + lax.axis_index("c")
        base = wid * b_per_w
        pltpu.sync_copy(idx_hbm.at[pl.ds(base, b_per_w)], idx_v)
        pltpu.async_copy(table_hbm.at[idx_v], rows_v, sem).wait()
        pltpu.sync_copy(rows_v, out_hbm.at[pl.ds(base, b_per_w)])

    return gk(table, idx)


def kernel(snip_feature, seg_lens, params):
    wb0 = _block_diag(params['w_b'][:, :, 0], 4)
    wb1 = _block_diag(params['w_b'][:, :, 1], 4)
    wb2 = _block_diag(params['w_b'][:, :, 2], 4)
    bb = params['b_b'].reshape(1, C)

    valid = (jnp.arange(T)[None, :] < seg_lens[:, None]).astype(F32)
    valid = valid.reshape(B, 1, T)

    g1 = _prep_block(params['g1'])
    g2 = _prep_block(params['g2'])
    eye = jnp.eye(T, dtype=F32)

    tpi1, ya1, ybp1, idx1 = _kp1(eye, snip_feature, valid, wb0, wb1, wb2,
                                 bb, g1)
    yg1 = _sc_gather(ya1.reshape(B * T, 128), idx1.reshape(-1))
    tpi2, ya2, ybp2, idx2 = _kp2(
        yg1, ybp1, tpi1, valid, g1[7], g1[8], g1[13], g2)
    yg2 = _sc_gather(ya2.reshape(B * T, 128), idx2.reshape(-1))
    return _kp3(eye, yg2, ybp2, tpi2, g2[7], g2[8], g2[13])
